# baseline (speedup 1.0000x reference)
import functools

import jax
import jax.numpy as jnp
from jax import lax
from jax.experimental import pallas as pl
from jax.experimental.pallas import tpu as pltpu
from jax.experimental.pallas import tpu_sc as plsc

_N_ATOMS = 256
_N_ANGLES = 254
_BATCH = 16384

_NC = 2
_NS = 16
_ABLK = 32
_LANES = 16
_ILV = 4

_GA = 8
_GC = (_NC * _NS) // _GA
_CPW = 128
_SC_COLS = _GC * _CPW
_TC_COLS = _BATCH - _SC_COLS
_CB = 2048

_APW = _N_ATOMS // _GA
_STAGE = 40
_BUF_ROWS = 48


def _rsqrt16(p):
    i = lax.bitcast_convert_type(p, jnp.int32)
    i = jnp.int32(0x5F3759DF) - (i >> 1)
    y = lax.bitcast_convert_type(i, jnp.float32)
    nh = p * jnp.float32(-0.5)
    for _ in range(2):
        y = y * (jnp.float32(1.5) + nh * y * y)
    return y


def _compute_block(in_v, out_v, a0):
    @plsc.parallel_loop(0, _CPW // (_ILV * _LANES))
    def j_body(j):
        col = pl.multiple_of(j * (_ILV * _LANES), _ILV * _LANES)
        cols = [col + k * _LANES for k in range(_ILV)]

        def ld(a, c, k):
            return in_v[c, a0 + a, pl.ds(cols[k], _LANES)]

        g0 = [[ld(0, c, k) for c in range(3)] for k in range(_ILV)]
        g1 = [[ld(1, c, k) for c in range(3)] for k in range(_ILV)]
        for t in range(_ABLK):
            g2 = [[ld(t + 2, c, k) for c in range(3)] for k in range(_ILV)]
            for k in range(_ILV):
                v1 = [g0[k][c] - g1[k][c] for c in range(3)]
                v2 = [g2[k][c] - g1[k][c] for c in range(3)]
                dot = v1[0] * v2[0] + v1[1] * v2[1] + v1[2] * v2[2]
                n1 = v1[0] * v1[0] + v1[1] * v1[1] + v1[2] * v1[2]
                n2 = v2[0] * v2[0] + v2[1] * v2[1] + v2[2] * v2[2]
                out_v[t, pl.ds(cols[k], _LANES)] = dot * _rsqrt16(n1 * n2)
            g0, g1 = g1, g2


def _sc_body(xt_hbm, o_hbm, in_v, out_v):
    wid = lax.axis_index("s") * _NC + lax.axis_index("c")
    g_a = wid % _GA
    g_c = wid // _GA
    base = _TC_COLS + g_c * _CPW
    row0 = pl.multiple_of(
        jnp.minimum(_APW * g_a, _N_ATOMS - _STAGE), 8)
    delta = _APW * g_a - row0

    pltpu.sync_copy(
        xt_hbm.at[:, pl.ds(row0, _STAGE), pl.ds(base, _CPW)],
        in_v.at[:, pl.ds(0, _STAGE)])

    def blk_body(blk, carry):
        a0 = pl.multiple_of(delta + _ABLK * blk, 8)
        _compute_block(in_v, out_v, a0)
        out_row = pl.multiple_of(_APW * g_a + _ABLK * blk, 8)
        pltpu.sync_copy(
            out_v, o_hbm.at[pl.ds(out_row, _ABLK), pl.ds(g_c * _CPW, _CPW)])
        return carry

    lax.fori_loop(0, _APW // _ABLK, blk_body, 0)


def _sc_kernel(xt):
    mesh = plsc.VectorSubcoreMesh(
        core_axis_name="c", subcore_axis_name="s", num_cores=_NC)
    run = functools.partial(
        pl.kernel,
        out_type=jax.ShapeDtypeStruct((_N_ATOMS, _SC_COLS), jnp.float32),
        mesh=mesh,
        scratch_types=[
            pltpu.VMEM((3, _BUF_ROWS, _CPW), jnp.float32),
            pltpu.VMEM((_ABLK, _CPW), jnp.float32),
        ],
    )(_sc_body)
    return run(xt)


def _tc_body(x_ref, o_ref):
    xs = [x_ref[c] for c in range(3)]
    d = [xc[0:_N_ANGLES + 1] - xc[1:_N_ANGLES + 2] for xc in xs]
    e = [dc * dc for dc in d]
    m = [d[c][0:_N_ANGLES] * d[c][1:_N_ANGLES + 1] for c in range(3)]
    dot = -(m[0] + m[1] + m[2])
    n1 = e[0][0:_N_ANGLES] + e[1][0:_N_ANGLES] + e[2][0:_N_ANGLES]
    n2 = (e[0][1:_N_ANGLES + 1] + e[1][1:_N_ANGLES + 1]
          + e[2][1:_N_ANGLES + 1])
    o_ref[...] = dot * jax.lax.rsqrt(n1 * n2)


def _tc_kernel(xt):
    tcb = 3968
    return pl.pallas_call(
        _tc_body,
        grid=(_TC_COLS // tcb,),
        in_specs=[pl.BlockSpec((3, _N_ATOMS, tcb), lambda i: (0, 0, i))],
        out_specs=pl.BlockSpec((_N_ANGLES, tcb), lambda i: (0, i)),
        out_shape=jax.ShapeDtypeStruct((_N_ANGLES, _BATCH), jnp.float32),
    )(xt)


def kernel(input):
    xt = jnp.transpose(input, (1, 0, 2))
    sc_out = _sc_kernel(xt)
    tc_out = _tc_kernel(xt)
    return lax.dynamic_update_slice(
        tc_out, sc_out[:_N_ANGLES], (0, _TC_COLS))


# speedup vs baseline: 1.0060x; 1.0060x over previous
import functools

import jax
import jax.numpy as jnp
from jax import lax
from jax.experimental import pallas as pl
from jax.experimental.pallas import tpu as pltpu
from jax.experimental.pallas import tpu_sc as plsc

_N_ATOMS = 256
_N_ANGLES = 254
_BATCH = 16384

_NC = 2
_NS = 16
_ABLK = 32
_LANES = 16
_ILV = 4

_GA = 4
_GC = (_NC * _NS) // _GA
_CPW = 128
_SC_COLS = _GC * _CPW
_TC_COLS = _BATCH - _SC_COLS
_APW = _N_ATOMS // _GA
_STAGE = 72
_BUF_ROWS = 80


def _rsqrt16(p):
    i = lax.bitcast_convert_type(p, jnp.int32)
    i = jnp.int32(0x5F3759DF) - (i >> 1)
    y = lax.bitcast_convert_type(i, jnp.float32)
    nh = p * jnp.float32(-0.5)
    for _ in range(2):
        y = y * (jnp.float32(1.5) + nh * y * y)
    return y


def _compute_block(in_v, out_v, a0):
    @plsc.parallel_loop(0, _CPW // (_ILV * _LANES))
    def j_body(j):
        col = pl.multiple_of(j * (_ILV * _LANES), _ILV * _LANES)
        cols = [col + k * _LANES for k in range(_ILV)]

        def ld(a, c, k):
            return in_v[c, a0 + a, pl.ds(cols[k], _LANES)]

        g0 = [[ld(0, c, k) for c in range(3)] for k in range(_ILV)]
        g1 = [[ld(1, c, k) for c in range(3)] for k in range(_ILV)]
        for t in range(_ABLK):
            g2 = [[ld(t + 2, c, k) for c in range(3)] for k in range(_ILV)]
            for k in range(_ILV):
                v1 = [g0[k][c] - g1[k][c] for c in range(3)]
                v2 = [g2[k][c] - g1[k][c] for c in range(3)]
                dot = v1[0] * v2[0] + v1[1] * v2[1] + v1[2] * v2[2]
                n1 = v1[0] * v1[0] + v1[1] * v1[1] + v1[2] * v1[2]
                n2 = v2[0] * v2[0] + v2[1] * v2[1] + v2[2] * v2[2]
                out_v[t, pl.ds(cols[k], _LANES)] = dot * _rsqrt16(n1 * n2)
            g0, g1 = g1, g2


def _sc_body(xt_hbm, o_hbm, in_v, out_v):
    wid = lax.axis_index("s") * _NC + lax.axis_index("c")
    g_a = wid % _GA
    g_c = wid // _GA
    base = _TC_COLS + g_c * _CPW
    row0 = pl.multiple_of(
        jnp.minimum(_APW * g_a, _N_ATOMS - _STAGE), 8)
    delta = _APW * g_a - row0

    pltpu.sync_copy(
        xt_hbm.at[:, pl.ds(row0, _STAGE), pl.ds(base, _CPW)],
        in_v.at[:, pl.ds(0, _STAGE)])

    def blk_body(blk, carry):
        a0 = pl.multiple_of(delta + _ABLK * blk, 8)
        _compute_block(in_v, out_v, a0)
        out_row = pl.multiple_of(_APW * g_a + _ABLK * blk, 8)
        pltpu.sync_copy(
            out_v, o_hbm.at[pl.ds(out_row, _ABLK), pl.ds(g_c * _CPW, _CPW)])
        return carry

    lax.fori_loop(0, _APW // _ABLK, blk_body, 0)


def _sc_kernel(xt):
    mesh = plsc.VectorSubcoreMesh(
        core_axis_name="c", subcore_axis_name="s", num_cores=_NC)
    run = functools.partial(
        pl.kernel,
        out_type=jax.ShapeDtypeStruct((_N_ATOMS, _SC_COLS), jnp.float32),
        mesh=mesh,
        scratch_types=[
            pltpu.VMEM((3, _BUF_ROWS, _CPW), jnp.float32),
            pltpu.VMEM((_ABLK, _CPW), jnp.float32),
        ],
    )(_sc_body)
    return run(xt)


def _tc_body(x_ref, o_ref):
    xs = [x_ref[c] for c in range(3)]
    d = [xc[0:_N_ANGLES + 1] - xc[1:_N_ANGLES + 2] for xc in xs]
    e = [dc * dc for dc in d]
    m = [d[c][0:_N_ANGLES] * d[c][1:_N_ANGLES + 1] for c in range(3)]
    dot = -(m[0] + m[1] + m[2])
    n1 = e[0][0:_N_ANGLES] + e[1][0:_N_ANGLES] + e[2][0:_N_ANGLES]
    n2 = (e[0][1:_N_ANGLES + 1] + e[1][1:_N_ANGLES + 1]
          + e[2][1:_N_ANGLES + 1])
    o_ref[...] = dot * jax.lax.rsqrt(n1 * n2)


def _tc_kernel(xt):
    tcb = 3072
    return pl.pallas_call(
        _tc_body,
        grid=(_TC_COLS // tcb,),
        in_specs=[pl.BlockSpec((3, _N_ATOMS, tcb), lambda i: (0, 0, i))],
        out_specs=pl.BlockSpec((_N_ANGLES, tcb), lambda i: (0, i)),
        out_shape=jax.ShapeDtypeStruct((_N_ANGLES, _BATCH), jnp.float32),
    )(xt)


def kernel(input):
    xt = jnp.transpose(input, (1, 0, 2))
    sc_out = _sc_kernel(xt)
    tc_out = _tc_kernel(xt)
    return lax.dynamic_update_slice(
        tc_out, sc_out[:_N_ANGLES], (0, _TC_COLS))
